# baseline (speedup 1.0000x reference)
import functools

import jax
import jax.numpy as jnp
from jax import lax
from jax.experimental import pallas as pl
from jax.experimental.pallas import tpu as pltpu
from jax.experimental.pallas import tpu_sc as plsc

L = 16
NC = 2
NS = 16
NW = NC * NS
B = 16384
K = 16
BPW = B // NW
GROUPS = BPW // L


def _body(uid_hbm, iid_hbm, uemb_hbm, ubias_hbm, iemb_hbm, ibias_hbm,
          bias_hbm, out_hbm,
          uidx_v, iidx_v, urows_v, irows_v, ub_v, ib_v, bias_v, out_v, sem):
    wid = lax.axis_index("s") * NC + lax.axis_index("c")
    base = wid * BPW

    pltpu.sync_copy(uid_hbm.at[pl.ds(base, BPW)], uidx_v)
    pltpu.sync_copy(iid_hbm.at[pl.ds(base, BPW)], iidx_v)

    cps = [
        pltpu.async_copy(uemb_hbm.at[uidx_v], urows_v, sem),
        pltpu.async_copy(iemb_hbm.at[iidx_v], irows_v, sem),
        pltpu.async_copy(ubias_hbm.at[uidx_v], ub_v, sem),
        pltpu.async_copy(ibias_hbm.at[iidx_v], ib_v, sem),
    ]
    pltpu.sync_copy(bias_hbm, bias_v)
    for cp in cps:
        cp.wait()

    bvec = bias_v[...]

    lane = lax.iota(jnp.int32, L)

    def group(g, carry):
        rb = g * L
        acc = ub_v[pl.ds(rb, L)] + ib_v[pl.ds(rb, L)] + bvec
        for r in range(L):
            prod = urows_v[rb + r] * irows_v[rb + r]
            s = jnp.sum(prod)
            acc = jnp.where(lane == r, acc + s, acc)
        out_v[pl.ds(rb, L)] = acc
        return carry

    lax.fori_loop(0, GROUPS, group, 0)

    pltpu.sync_copy(out_v, out_hbm.at[pl.ds(base, BPW)])


_mesh = plsc.VectorSubcoreMesh(core_axis_name="c", subcore_axis_name="s")

_sc_call = functools.partial(
    pl.kernel,
    out_type=jax.ShapeDtypeStruct((B,), jnp.float32),
    mesh=_mesh,
    compiler_params=pltpu.CompilerParams(needs_layout_passes=False,
                                         use_tc_tiling_on_sc=False),
    scratch_types=[
        pltpu.VMEM((BPW,), jnp.int32),
        pltpu.VMEM((BPW,), jnp.int32),
        pltpu.VMEM((BPW, K), jnp.float32),
        pltpu.VMEM((BPW, K), jnp.float32),
        pltpu.VMEM((BPW,), jnp.float32),
        pltpu.VMEM((BPW,), jnp.float32),
        pltpu.VMEM((L,), jnp.float32),
        pltpu.VMEM((BPW,), jnp.float32),
        pltpu.SemaphoreType.DMA,
    ],
)(_body)


@jax.jit
def kernel(user_id, item_id, user_emb, user_bias, item_emb, item_bias, bias):
    bias16 = jnp.broadcast_to(bias.astype(jnp.float32), (L,))
    return _sc_call(user_id.astype(jnp.int32), item_id.astype(jnp.int32),
                    user_emb, user_bias, item_emb, item_bias, bias16)
